# baseline (speedup 1.0000x reference)
import functools

import jax
import jax.numpy as jnp
from jax import lax
from jax.experimental import pallas as pl
from jax.experimental.pallas import tpu as pltpu
from jax.experimental.pallas import tpu_sc as plsc

_M, _N, _B, _U = 8, 128, 256, 64
_LANES = 128
_ROWS = _N * _B * _U // _LANES
_ROW_BLOCK = 1024
_LOG2E = 1.4426950408889634


@functools.cache
def _make_gather_coefs():
    mesh = plsc.VectorSubcoreMesh(core_axis_name="c", subcore_axis_name="s")

    @functools.partial(
        pl.kernel,
        mesh=mesh,
        out_type=jax.ShapeDtypeStruct((_M, 8, _LANES), jnp.float32),
        scratch_types=[
            pltpu.VMEM((_U,), jnp.int32),
            pltpu.VMEM((128,), jnp.float32),
            pltpu.VMEM((_LANES,), jnp.float32),
        ],
        compiler_params=pltpu.CompilerParams(needs_layout_passes=False),
    )
    def _gather_coefs(mask_hbm, eta_hbm, out_hbm, mask_v, eta_v, row_v):
        wid = lax.axis_index("s") * 2 + lax.axis_index("c")
        m = wid // 4
        k = wid % 4
        pltpu.sync_copy(mask_hbm.at[m], mask_v)
        pltpu.sync_copy(eta_hbm, eta_v)
        scale = jnp.where(k == 3, jnp.float32(_LOG2E), jnp.float32(1.0))
        for g in range(4):
            idx = mask_v[pl.ds(g * 16, 16)] * 8 + k
            vals = plsc.load_gather(eta_v, [idx]) * scale
            row_v[pl.ds(g * 16, 16)] = vals
            row_v[pl.ds(64 + g * 16, 16)] = vals
        pltpu.sync_copy(row_v, out_hbm.at[m, k])
        zeros = jnp.zeros((16,), jnp.float32)
        for g in range(8):
            row_v[pl.ds(g * 16, 16)] = zeros
        pltpu.sync_copy(row_v, out_hbm.at[m, k + 4])

    return _gather_coefs


def _sigmoid_body(c_ref, z_ref, o_ref):
    c = c_ref[0]
    x = z_ref[0]
    e0 = c[0:1, :]
    e1 = c[1:2, :]
    e2 = c[2:3, :]
    e3 = c[3:4, :]
    o_ref[0] = e0 + e1 / (1.0 + jnp.exp2((e2 - x) * e3))


def kernel(z, Mask, eta_fault):
    mask_i32 = Mask.astype(jnp.int32)
    eta_pad = jnp.zeros((16, 8), jnp.float32).at[:15, :4].set(eta_fault).reshape(128)
    coefs = _make_gather_coefs()(mask_i32, eta_pad)

    z3 = z.reshape(_M, _ROWS, _LANES)
    out3 = pl.pallas_call(
        _sigmoid_body,
        grid=(_M, _ROWS // _ROW_BLOCK),
        in_specs=[
            pl.BlockSpec((1, 8, _LANES), lambda m, j: (m, 0, 0)),
            pl.BlockSpec((1, _ROW_BLOCK, _LANES), lambda m, j: (m, j, 0)),
        ],
        out_specs=pl.BlockSpec((1, _ROW_BLOCK, _LANES), lambda m, j: (m, j, 0)),
        out_shape=jax.ShapeDtypeStruct((_M, _ROWS, _LANES), jnp.float32),
        compiler_params=pltpu.CompilerParams(
            dimension_semantics=("parallel", "parallel"),
        ),
    )(coefs, z3)
    return out3.reshape(_M, _N, _B, _U)


# speedup vs baseline: 4.4269x; 4.4269x over previous
import functools

import jax
import jax.numpy as jnp
from jax import lax
from jax.experimental import pallas as pl
from jax.experimental.pallas import tpu as pltpu
from jax.experimental.pallas import tpu_sc as plsc

_M, _N, _B, _U = 8, 128, 256, 64
_N_BLOCK = 16


@functools.cache
def _make_gather_coefs():
    mesh = plsc.VectorSubcoreMesh(core_axis_name="c", subcore_axis_name="s")

    @functools.partial(
        pl.kernel,
        mesh=mesh,
        out_type=jax.ShapeDtypeStruct((_M, 4, _U, 128), jnp.float32),
        scratch_types=[
            pltpu.VMEM((_U,), jnp.int32),
            pltpu.VMEM((128,), jnp.float32),
            pltpu.VMEM((16,), jnp.float32),
            pltpu.VMEM((_U, 128), jnp.float32),
        ],
        compiler_params=pltpu.CompilerParams(needs_layout_passes=False),
    )
    def _gather_coefs(mask_hbm, eta_hbm, out_hbm, mask_v, eta_v, vals_v, plane_v):
        wid = lax.axis_index("s") * 2 + lax.axis_index("c")
        m = wid // 4
        k = wid % 4
        pltpu.sync_copy(mask_hbm.at[m], mask_v)
        pltpu.sync_copy(eta_hbm, eta_v)
        half = jnp.float32(0.5)
        for g in range(4):
            idx8 = mask_v[pl.ds(g * 16, 16)] * 8
            v_self = plsc.load_gather(eta_v, [idx8 + k])
            v_e1 = plsc.load_gather(eta_v, [idx8 + 1])
            vals = jnp.where(
                k == 0,
                v_self + half * v_e1,
                jnp.where(k == 2, v_self, half * v_self),
            )
            vals_v[...] = vals
            for j in range(16):
                u = g * 16 + j
                row = plsc.load_gather(vals_v, [jnp.zeros((16,), jnp.int32) + j])
                for c in range(8):
                    plane_v[u, pl.ds(c * 16, 16)] = row
        pltpu.sync_copy(plane_v, out_hbm.at[m, k])

    return _gather_coefs


def _tanh_body(c_ref, z_ref, o_ref):
    c = c_ref[0]
    x = z_ref[0]
    c0 = c[0, :, 0:1]
    c1 = c[1, :, 0:1]
    c2 = c[2, :, 0:1]
    c3 = c[3, :, 0:1]
    o_ref[0] = c0 + c1 * jnp.tanh((x - c2) * c3)


def kernel(z, Mask, eta_fault):
    mask_i32 = Mask.astype(jnp.int32)
    eta_pad = jnp.zeros((16, 8), jnp.float32).at[:15, :4].set(eta_fault).reshape(128)
    planes = _make_gather_coefs()(mask_i32, eta_pad)

    zt = jnp.transpose(z, (0, 1, 3, 2))
    out_t = pl.pallas_call(
        _tanh_body,
        grid=(_M, _N // _N_BLOCK),
        in_specs=[
            pl.BlockSpec((1, 4, _U, 128), lambda m, j: (m, 0, 0, 0)),
            pl.BlockSpec((1, _N_BLOCK, _U, _B), lambda m, j: (m, j, 0, 0)),
        ],
        out_specs=pl.BlockSpec((1, _N_BLOCK, _U, _B), lambda m, j: (m, j, 0, 0)),
        out_shape=jax.ShapeDtypeStruct((_M, _N, _U, _B), jnp.float32),
        compiler_params=pltpu.CompilerParams(
            dimension_semantics=("parallel", "parallel"),
        ),
    )(planes, zt)
    return jnp.transpose(out_t, (0, 1, 3, 2))
